# baseline (speedup 1.0000x reference)
import jax
import jax.numpy as jnp
from jax.experimental import pallas as pl
from jax.experimental.pallas import tpu as pltpu

_D_MODEL = 2048
_D_SPACE = 64
_N_OUT = 1024
_TB1 = 1024
_T_HALF = _TB1 // 2
_TB2 = 2048


def _proj_kernel(xa_ref, xb_ref, w_ref, b_ref, h_ref):
    w = w_ref[...]
    bias = b_ref[...]
    h_ref[:_T_HALF, :] = (
        jnp.dot(xa_ref[...], w, preferred_element_type=jnp.float32) + bias
    )
    h_ref[_T_HALF:, :] = (
        jnp.dot(xb_ref[...], w, preferred_element_type=jnp.float32) + bias
    )


def _logits_kernel(h_ref, emb_ref, out_ref):
    emb = emb_ref[...]
    norm = jnp.sqrt(jnp.sum(emb * emb, axis=1, keepdims=True))
    embn = emb / jnp.maximum(norm, 1e-12)
    out_ref[...] = jax.lax.dot_general(
        h_ref[...], embn, (((1,), (1,)), ((), ())),
        preferred_element_type=jnp.float32,
    )


def kernel(x, W, b, neuron_emb):
    B, S, _ = x.shape
    tokens = B * S
    x2 = x.reshape(tokens, _D_MODEL)
    emb = neuron_emb[:_N_OUT]
    b2 = b.reshape(1, _D_SPACE)
    h = pl.pallas_call(
        _proj_kernel,
        grid=(tokens // _TB1,),
        in_specs=[
            pl.BlockSpec((_T_HALF, _D_MODEL), lambda i: (2 * i, 0)),
            pl.BlockSpec((_T_HALF, _D_MODEL), lambda i: (2 * i + 1, 0)),
            pl.BlockSpec((_D_MODEL, _D_SPACE), lambda i: (0, 0)),
            pl.BlockSpec((1, _D_SPACE), lambda i: (0, 0)),
        ],
        out_specs=pl.BlockSpec((_TB1, _D_SPACE), lambda i: (i, 0)),
        out_shape=jax.ShapeDtypeStruct((tokens, _D_SPACE), jnp.float32),
        compiler_params=pltpu.CompilerParams(
            dimension_semantics=("parallel",),
        ),
    )(x2, x2, W, b2)
    out = pl.pallas_call(
        _logits_kernel,
        grid=(tokens // _TB2,),
        in_specs=[
            pl.BlockSpec((_TB2, _D_SPACE), lambda i: (i, 0)),
            pl.BlockSpec((_N_OUT, _D_SPACE), lambda i: (0, 0)),
        ],
        out_specs=pl.BlockSpec((_TB2, _N_OUT), lambda i: (i, 0)),
        out_shape=jax.ShapeDtypeStruct((tokens, _N_OUT), jnp.float32),
        compiler_params=pltpu.CompilerParams(
            dimension_semantics=("parallel",),
        ),
    )(h, emb)
    return out.reshape(B, S, _N_OUT)


# speedup vs baseline: 1.0507x; 1.0507x over previous
import jax
import jax.numpy as jnp
from jax.experimental import pallas as pl
from jax.experimental.pallas import tpu as pltpu

_D_MODEL = 2048
_D_SPACE = 64
_N_OUT = 1024
_TOKEN_BLOCK = 1024
_T_HALF = _TOKEN_BLOCK // 2


def _router_kernel(xa_ref, xb_ref, w_ref, b_ref, emb_ref, out_ref):
    w = w_ref[...]
    bias = b_ref[...]
    emb = emb_ref[...]
    norm = jnp.sqrt(jnp.sum(emb * emb, axis=1, keepdims=True))
    embn = emb / jnp.maximum(norm, 1e-12)
    ha = jnp.dot(xa_ref[...], w, preferred_element_type=jnp.float32) + bias
    hb = jnp.dot(xb_ref[...], w, preferred_element_type=jnp.float32) + bias
    out_ref[:_T_HALF, :] = jax.lax.dot_general(
        ha, embn, (((1,), (1,)), ((), ())), preferred_element_type=jnp.float32
    )
    out_ref[_T_HALF:, :] = jax.lax.dot_general(
        hb, embn, (((1,), (1,)), ((), ())), preferred_element_type=jnp.float32
    )


def kernel(x, W, b, neuron_emb):
    B, S, _ = x.shape
    tokens = B * S
    x2 = x.reshape(tokens, _D_MODEL)
    emb = neuron_emb[:_N_OUT]
    b2 = b.reshape(1, _D_SPACE)
    grid = (tokens // _TOKEN_BLOCK,)
    out = pl.pallas_call(
        _router_kernel,
        grid=grid,
        in_specs=[
            pl.BlockSpec((_T_HALF, _D_MODEL), lambda i: (2 * i, 0)),
            pl.BlockSpec((_T_HALF, _D_MODEL), lambda i: (2 * i + 1, 0)),
            pl.BlockSpec((_D_MODEL, _D_SPACE), lambda i: (0, 0)),
            pl.BlockSpec((1, _D_SPACE), lambda i: (0, 0)),
            pl.BlockSpec((_N_OUT, _D_SPACE), lambda i: (0, 0)),
        ],
        out_specs=pl.BlockSpec((_TOKEN_BLOCK, _N_OUT), lambda i: (i, 0)),
        out_shape=jax.ShapeDtypeStruct((tokens, _N_OUT), jnp.float32),
        compiler_params=pltpu.CompilerParams(
            dimension_semantics=("parallel",),
        ),
    )(x2, x2, W, b2, emb)
    return out.reshape(B, S, _N_OUT)


# speedup vs baseline: 1.0971x; 1.0442x over previous
import jax
import jax.numpy as jnp
from jax.experimental import pallas as pl
from jax.experimental.pallas import tpu as pltpu

_D_MODEL = 2048
_D_SPACE = 64
_N_OUT = 1024
_TOKEN_BLOCK = 2048
_T_HALF = _TOKEN_BLOCK // 2


def _router_kernel(xa_ref, xb_ref, w_ref, b_ref, emb_ref, out_ref):
    w = w_ref[...]
    bias = b_ref[...]
    emb = emb_ref[...]
    norm = jnp.sqrt(jnp.sum(emb * emb, axis=1, keepdims=True))
    embn = emb / jnp.maximum(norm, 1e-12)
    ha = jnp.dot(xa_ref[...], w, preferred_element_type=jnp.float32) + bias
    hb = jnp.dot(xb_ref[...], w, preferred_element_type=jnp.float32) + bias
    out_ref[:_T_HALF, :] = jax.lax.dot_general(
        ha, embn, (((1,), (1,)), ((), ())), preferred_element_type=jnp.float32
    )
    out_ref[_T_HALF:, :] = jax.lax.dot_general(
        hb, embn, (((1,), (1,)), ((), ())), preferred_element_type=jnp.float32
    )


def kernel(x, W, b, neuron_emb):
    B, S, _ = x.shape
    tokens = B * S
    x2 = x.reshape(tokens, _D_MODEL)
    emb = neuron_emb[:_N_OUT]
    b2 = b.reshape(1, _D_SPACE)
    grid = (tokens // _TOKEN_BLOCK,)
    out = pl.pallas_call(
        _router_kernel,
        grid=grid,
        in_specs=[
            pl.BlockSpec((_T_HALF, _D_MODEL), lambda i: (2 * i, 0)),
            pl.BlockSpec((_T_HALF, _D_MODEL), lambda i: (2 * i + 1, 0)),
            pl.BlockSpec((_D_MODEL, _D_SPACE), lambda i: (0, 0)),
            pl.BlockSpec((1, _D_SPACE), lambda i: (0, 0)),
            pl.BlockSpec((_N_OUT, _D_SPACE), lambda i: (0, 0)),
        ],
        out_specs=pl.BlockSpec((_TOKEN_BLOCK, _N_OUT), lambda i: (i, 0)),
        out_shape=jax.ShapeDtypeStruct((tokens, _N_OUT), jnp.float32),
        compiler_params=pltpu.CompilerParams(
            dimension_semantics=("arbitrary",),
        ),
    )(x2, x2, W, b2, emb)
    return out.reshape(B, S, _N_OUT)


# speedup vs baseline: 1.1046x; 1.0068x over previous
import jax
import jax.numpy as jnp
from jax.experimental import pallas as pl
from jax.experimental.pallas import tpu as pltpu

_D_MODEL = 2048
_D_SPACE = 64
_N_OUT = 1024
_TOKEN_BLOCK = 2048
_T_HALF = _TOKEN_BLOCK // 2


def _router_kernel(xa_ref, xb_ref, w_ref, b_ref, emb_ref, out_ref):
    w = w_ref[...]
    bias = b_ref[...]
    emb = emb_ref[...]
    norm = jnp.sqrt(jnp.sum(emb * emb, axis=1, keepdims=True))
    embn = emb / jnp.maximum(norm, 1e-12)
    ha = jnp.dot(xa_ref[...], w, preferred_element_type=jnp.float32) + bias
    hb = jnp.dot(xb_ref[...], w, preferred_element_type=jnp.float32) + bias
    out_ref[:_T_HALF, :] = jax.lax.dot_general(
        ha, embn, (((1,), (1,)), ((), ())), preferred_element_type=jnp.float32
    )
    out_ref[_T_HALF:, :] = jax.lax.dot_general(
        hb, embn, (((1,), (1,)), ((), ())), preferred_element_type=jnp.float32
    )


def kernel(x, W, b, neuron_emb):
    B, S, _ = x.shape
    tokens = B * S
    x2 = x.reshape(tokens, _D_MODEL)
    emb = neuron_emb[:_N_OUT]
    b2 = b.reshape(1, _D_SPACE)
    grid = (tokens // _TOKEN_BLOCK,)
    out = pl.pallas_call(
        _router_kernel,
        grid=grid,
        in_specs=[
            pl.BlockSpec((_T_HALF, _D_MODEL), lambda i: (2 * i, 0)),
            pl.BlockSpec((_T_HALF, _D_MODEL), lambda i: (2 * i + 1, 0)),
            pl.BlockSpec((_D_MODEL, _D_SPACE), lambda i: (0, 0)),
            pl.BlockSpec((1, _D_SPACE), lambda i: (0, 0)),
            pl.BlockSpec((_N_OUT, _D_SPACE), lambda i: (0, 0)),
        ],
        out_specs=pl.BlockSpec((_TOKEN_BLOCK, _N_OUT), lambda i: (i, 0)),
        out_shape=jax.ShapeDtypeStruct((tokens, _N_OUT), jnp.float32),
        compiler_params=pltpu.CompilerParams(
            dimension_semantics=("parallel",),
        ),
    )(x2, x2, W, b2, emb)
    return out.reshape(B, S, _N_OUT)
